# Initial kernel scaffold; baseline (speedup 1.0000x reference)
import jax
import jax.numpy as jnp
from jax.experimental import pallas as pl


def kernel(token_ids, token_table, pos_table, ln_gamma, ln_beta):
    raise NotImplementedError("write your pallas kernel here")


# speedup vs baseline: 1.5227x; 1.5227x over previous
import functools

import jax
import jax.numpy as jnp
from jax import lax
from jax.experimental import pallas as pl
from jax.experimental.pallas import tpu as pltpu
from jax.experimental.pallas import tpu_sc as plsc

VOCAB = 100000
EMBED_DIM = 64
MAX_SEQ_LEN = 512
BATCH = 1024
SEQ_LEN = 200
LN_EPS = 1e-5

NUM_CORES = 2
NUM_SUBCORES = 16
LANES = 16
NUM_WORKERS = NUM_CORES * NUM_SUBCORES
TOTAL = BATCH * SEQ_LEN
PER_WORKER = TOTAL // NUM_WORKERS
CHUNK = 128
CHUNKS_PER_WORKER = PER_WORKER // CHUNK
VPR = EMBED_DIM // LANES


_GATHER_DNUMS = lax.GatherDimensionNumbers(
    offset_dims=(), collapsed_slice_dims=(0,), start_index_map=(0,))


def _lane_permute(v, p):
    return lax.gather(v, p[:, None], _GATHER_DNUMS, slice_sizes=(1,),
                      mode=lax.GatherScatterMode.PROMISE_IN_BOUNDS)


def _splat_sum(v, perms):
    for p in perms:
        v = v + _lane_permute(v, p)
    return v


def _ln_body(ids_hbm, token_table, pos_hbm, gamma_hbm, beta_hbm, out_hbm,
             idx_buf, rows_buf, pos_slab, gamma_v, beta_v, sem):
    wid = lax.axis_index("s") * NUM_CORES + lax.axis_index("c")
    base = wid * PER_WORKER

    pltpu.sync_copy(pos_hbm.at[pl.ds(0, SEQ_LEN)], pos_slab)
    pltpu.sync_copy(gamma_hbm, gamma_v)
    pltpu.sync_copy(beta_hbm, beta_v)

    iota = lax.iota(jnp.int32, LANES)
    perms = [iota ^ (1 << k) for k in range(4)]

    gammas = [gamma_v[pl.ds(k * LANES, LANES)] for k in range(VPR)]
    betas = [beta_v[pl.ds(k * LANES, LANES)] for k in range(VPR)]
    inv_d = jnp.float32(1.0 / EMBED_DIM)

    def chunk_body(c, carry):
        row0 = base + c * CHUNK
        pltpu.sync_copy(ids_hbm.at[pl.ds(row0, CHUNK)], idx_buf)
        pltpu.async_copy(token_table.at[idx_buf], rows_buf, sem).wait()

        def row_body(r, carry2):
            s = lax.rem(row0 + r, SEQ_LEN)
            e = [rows_buf[r, pl.ds(k * LANES, LANES)] +
                 pos_slab[s, pl.ds(k * LANES, LANES)] for k in range(VPR)]
            t = (e[0] + e[1]) + (e[2] + e[3])
            u = (e[0] * e[0] + e[1] * e[1]) + (e[2] * e[2] + e[3] * e[3])
            t = _splat_sum(t, perms)
            u = _splat_sum(u, perms)
            mean = t * inv_d
            var = u * inv_d - mean * mean + jnp.float32(LN_EPS)
            y = lax.bitcast_convert_type(
                jnp.int32(0x5F3759DF)
                - (lax.bitcast_convert_type(var, jnp.int32) >> 1),
                jnp.float32)
            half = var * jnp.float32(0.5)
            for _ in range(3):
                y = y * (jnp.float32(1.5) - half * y * y)
            scaled_mean = mean * y
            for k in range(VPR):
                rows_buf[r, pl.ds(k * LANES, LANES)] = (
                    (e[k] * y - scaled_mean) * gammas[k] + betas[k])
            return carry2

        lax.fori_loop(0, CHUNK, row_body, 0, unroll=2)
        pltpu.sync_copy(rows_buf, out_hbm.at[pl.ds(row0, CHUNK)])
        return carry

    lax.fori_loop(0, CHUNKS_PER_WORKER, chunk_body, 0)


@jax.jit
def _hybrid_embed(token_ids_flat, token_table, pos_table, ln_gamma, ln_beta):
    mesh = plsc.VectorSubcoreMesh(core_axis_name="c", subcore_axis_name="s",
                                  num_cores=NUM_CORES,
                                  num_subcores=NUM_SUBCORES)
    return pl.kernel(
        _ln_body,
        out_type=jax.ShapeDtypeStruct((TOTAL, EMBED_DIM), jnp.float32),
        mesh=mesh,
        scratch_types=[
            pltpu.VMEM((CHUNK,), jnp.int32),
            pltpu.VMEM((CHUNK, EMBED_DIM), jnp.float32),
            pltpu.VMEM((SEQ_LEN, EMBED_DIM), jnp.float32),
            pltpu.VMEM((EMBED_DIM,), jnp.float32),
            pltpu.VMEM((EMBED_DIM,), jnp.float32),
            pltpu.SemaphoreType.DMA,
        ],
        compiler_params=pltpu.CompilerParams(use_tc_tiling_on_sc=False),
    )(token_ids_flat, token_table, pos_table, ln_gamma, ln_beta)


def kernel(token_ids, token_table, pos_table, ln_gamma, ln_beta):
    ids_flat = token_ids.reshape(-1).astype(jnp.int32)
    out = _hybrid_embed(ids_flat, token_table, pos_table, ln_gamma, ln_beta)
    return out.reshape(BATCH, SEQ_LEN, EMBED_DIM)


# speedup vs baseline: 1.9086x; 1.2534x over previous
import jax
import jax.numpy as jnp
from jax import lax
from jax.experimental import pallas as pl
from jax.experimental.pallas import tpu as pltpu
from jax.experimental.pallas import tpu_sc as plsc

VOCAB = 100000
EMBED_DIM = 64
MAX_SEQ_LEN = 512
BATCH = 1024
SEQ_LEN = 200
LN_EPS = 1e-5

NUM_CORES = 2
NUM_SUBCORES = 16
LANES = 16
NUM_WORKERS = NUM_CORES * NUM_SUBCORES
TOTAL = BATCH * SEQ_LEN
PER_WORKER = TOTAL // NUM_WORKERS
CHUNK = 128
NUM_CHUNKS = PER_WORKER // CHUNK
VPR = EMBED_DIM // LANES

_GATHER_DNUMS = lax.GatherDimensionNumbers(
    offset_dims=(), collapsed_slice_dims=(0,), start_index_map=(0,))


def _lane_permute(v, p):
    return lax.gather(v, p[:, None], _GATHER_DNUMS, slice_sizes=(1,),
                      mode=lax.GatherScatterMode.PROMISE_IN_BOUNDS)


def _splat_sum(v, perms):
    for p in perms:
        v = v + _lane_permute(v, p)
    return v


def _ln_body(ids_hbm, token_table, pos_hbm, gamma_hbm, beta_hbm, out_hbm,
             idx_bufs, rows_bufs, out_bufs, pos_slab, gamma_v, beta_v,
             gsems, osems, isems):
    wid = lax.axis_index("s") * NUM_CORES + lax.axis_index("c")
    base = wid * PER_WORKER

    pltpu.sync_copy(pos_hbm.at[pl.ds(0, SEQ_LEN)], pos_slab)
    pltpu.sync_copy(gamma_hbm, gamma_v)
    pltpu.sync_copy(beta_hbm, beta_v)

    iota = lax.iota(jnp.int32, LANES)
    perms = [iota ^ (1 << k) for k in range(4)]

    gammas = [gamma_v[pl.ds(k * LANES, LANES)] for k in range(VPR)]
    betas = [beta_v[pl.ds(k * LANES, LANES)] for k in range(VPR)]
    inv_d = jnp.float32(1.0 / EMBED_DIM)

    def normalize_chunk(row0, rows_buf, out_buf):
        def row_body(r, carry):
            s = lax.rem(row0 + r, SEQ_LEN)
            e = [rows_buf[r, pl.ds(k * LANES, LANES)] +
                 pos_slab[s, pl.ds(k * LANES, LANES)] for k in range(VPR)]
            t = (e[0] + e[1]) + (e[2] + e[3])
            u = (e[0] * e[0] + e[1] * e[1]) + (e[2] * e[2] + e[3] * e[3])
            t = _splat_sum(t, perms)
            u = _splat_sum(u, perms)
            mean = t * inv_d
            var = u * inv_d - mean * mean + jnp.float32(LN_EPS)
            y = lax.bitcast_convert_type(
                jnp.int32(0x5F3759DF)
                - (lax.bitcast_convert_type(var, jnp.int32) >> 1),
                jnp.float32)
            half = var * jnp.float32(0.5)
            for _ in range(2):
                y = y * (jnp.float32(1.5) - half * y * y)
            scaled_mean = mean * y
            for k in range(VPR):
                out_buf[r, pl.ds(k * LANES, LANES)] = (
                    (e[k] * y - scaled_mean) * gammas[k] + betas[k])
            return carry

        lax.fori_loop(0, CHUNK, row_body, 0, unroll=4)

    def start_gather(c, slot):
        pltpu.async_copy(token_table.at[idx_bufs[slot]], rows_bufs[slot],
                         gsems[slot])

    for slot in range(2):
        pltpu.sync_copy(ids_hbm.at[pl.ds(base + slot * CHUNK, CHUNK)],
                        idx_bufs[slot])
        start_gather(slot, slot)

    def chunk_step(cc, carry):
        for k in range(2):
            row0 = base + (cc * 2 + k) * CHUNK
            prefetch = cc < (NUM_CHUNKS // 2) - 1

            pltpu.make_async_copy(token_table.at[idx_bufs[k]], rows_bufs[k],
                                  gsems[k]).wait()
            @pl.when(prefetch)
            def _():
                pltpu.async_copy(
                    ids_hbm.at[pl.ds(row0 + 2 * CHUNK, CHUNK)],
                    idx_bufs[k], isems[k])

            @pl.when(cc > 0)
            def _():
                pltpu.make_async_copy(
                    out_bufs[k], out_hbm.at[pl.ds(row0 - 2 * CHUNK, CHUNK)],
                    osems[k]).wait()

            normalize_chunk(row0, rows_bufs[k], out_bufs[k])

            pltpu.async_copy(out_bufs[k], out_hbm.at[pl.ds(row0, CHUNK)],
                             osems[k])
            @pl.when(prefetch)
            def _():
                pltpu.make_async_copy(
                    ids_hbm.at[pl.ds(row0 + 2 * CHUNK, CHUNK)],
                    idx_bufs[k], isems[k]).wait()
                start_gather(cc * 2 + k + 2, k)
        return carry

    lax.fori_loop(0, NUM_CHUNKS // 2, chunk_step, 0)

    for k in range(2):
        row0 = base + (NUM_CHUNKS - 2 + k) * CHUNK
        pltpu.make_async_copy(out_bufs[k], out_hbm.at[pl.ds(row0, CHUNK)],
                              osems[k]).wait()


@jax.jit
def _hybrid_embed(token_ids_flat, token_table, pos_table, ln_gamma, ln_beta):
    mesh = plsc.VectorSubcoreMesh(core_axis_name="c", subcore_axis_name="s",
                                  num_cores=NUM_CORES,
                                  num_subcores=NUM_SUBCORES)
    return pl.kernel(
        _ln_body,
        out_type=jax.ShapeDtypeStruct((TOTAL, EMBED_DIM), jnp.float32),
        mesh=mesh,
        scratch_types=[
            [pltpu.VMEM((CHUNK,), jnp.int32) for _ in range(2)],
            [pltpu.VMEM((CHUNK, EMBED_DIM), jnp.float32) for _ in range(2)],
            [pltpu.VMEM((CHUNK, EMBED_DIM), jnp.float32) for _ in range(2)],
            pltpu.VMEM((SEQ_LEN, EMBED_DIM), jnp.float32),
            pltpu.VMEM((EMBED_DIM,), jnp.float32),
            pltpu.VMEM((EMBED_DIM,), jnp.float32),
            [pltpu.SemaphoreType.DMA for _ in range(2)],
            [pltpu.SemaphoreType.DMA for _ in range(2)],
            [pltpu.SemaphoreType.DMA for _ in range(2)],
        ],
        compiler_params=pltpu.CompilerParams(use_tc_tiling_on_sc=False),
    )(token_ids_flat, token_table, pos_table, ln_gamma, ln_beta)


def kernel(token_ids, token_table, pos_table, ln_gamma, ln_beta):
    ids_flat = token_ids.reshape(-1).astype(jnp.int32)
    out = _hybrid_embed(ids_flat, token_table, pos_table, ln_gamma, ln_beta)
    return out.reshape(BATCH, SEQ_LEN, EMBED_DIM)
